# Initial kernel scaffold; baseline (speedup 1.0000x reference)
import jax
import jax.numpy as jnp
from jax.experimental import pallas as pl


def kernel(uvi, tables, W1, b1, W2, b2):
    raise NotImplementedError("write your pallas kernel here")


# speedup vs baseline: 183.5584x; 183.5584x over previous
import functools

import jax
import jax.numpy as jnp
import numpy as np
from jax import lax
from jax.experimental import pallas as pl
from jax.experimental.pallas import tpu as pltpu
from jax.experimental.pallas import tpu_sc as plsc

L = 8
T = 1 << 16
P1 = np.int32(np.uint32(2654435761))
P2 = np.int32(805459861)
NWORKERS = 32
NCHUNKS = NWORKERS // L
STRIPE = 2048
LANES = 16


def _sc_encode(n):
    chunk = n // NCHUNKS
    nstripes = chunk // STRIPE
    groups = STRIPE // LANES
    mesh = plsc.VectorSubcoreMesh(core_axis_name="c", subcore_axis_name="s")

    @functools.partial(
        pl.kernel,
        out_type=jax.ShapeDtypeStruct((2 * L * n,), jnp.float32),
        mesh=mesh,
        scratch_types=[
            pltpu.VMEM((T,), jnp.int32),
            pltpu.VMEM((STRIPE,), jnp.float32),
            pltpu.VMEM((STRIPE,), jnp.float32),
            pltpu.VMEM((STRIPE,), jnp.float32),
            pltpu.VMEM((STRIPE,), jnp.float32),
            pltpu.VMEM((STRIPE,), jnp.float32),
        ],
        compiler_params=pltpu.CompilerParams(use_tc_tiling_on_sc=False,
                                             needs_layout_passes=False),
    )
    def encode(uvi_t, ptab, out, tab_v, ux_v, uy_v, uz_v, e0_v, e1_v):
        wid = lax.axis_index("s") * 2 + lax.axis_index("c")
        level = wid % L
        chunk_id = wid // L
        res_i = lax.shift_left(np.int32(16), level)
        res_f = res_i.astype(jnp.float32)
        rp1 = res_i + 1
        is_dense = level < 2

        pltpu.sync_copy(ptab.at[pl.ds(level * T, T)], tab_v)

        def stripe_body(s, _):
            base = chunk_id * chunk + s * STRIPE
            pltpu.sync_copy(uvi_t.at[pl.ds(base, STRIPE)], ux_v)
            pltpu.sync_copy(uvi_t.at[pl.ds(n + base, STRIPE)], uy_v)
            pltpu.sync_copy(uvi_t.at[pl.ds(2 * n + base, STRIPE)], uz_v)

            def group_body(g, _):
                sl = pl.ds(g * LANES, LANES)
                x = ux_v[sl] * res_f
                y = uy_v[sl] * res_f
                z = uz_v[sl] * res_f
                ix = x.astype(jnp.int32)
                iy = y.astype(jnp.int32)
                iz = z.astype(jnp.int32)
                wx = x - ix.astype(jnp.float32)
                wy = y - iy.astype(jnp.float32)
                wz = z - iz.astype(jnp.float32)
                ixc = jnp.minimum(ix + 1, res_i)
                iyc = jnp.minimum(iy + 1, res_i)
                izc = jnp.minimum(iz + 1, res_i)
                hy = (iy * P1, iyc * P1)
                hz = (iz * P2, izc * P2)
                dyz = tuple(rp1 * (py + rp1 * pz)
                            for pz in (iz, izc) for py in (iy, iyc))
                px = (ix, ixc)
                sx = (1.0 - wx, wx)
                sxy = tuple(sy * s for s in (1.0 - wy, wy) for sy in sx)
                sz = (1.0 - wz, wz)
                acc0 = jnp.zeros((LANES,), jnp.float32)
                acc1 = jnp.zeros((LANES,), jnp.float32)
                for corner in range(8):
                    ox = corner & 1
                    oy = (corner >> 1) & 1
                    oz = (corner >> 2) & 1
                    hidx = (px[ox] ^ hy[oy] ^ hz[oz]) & np.int32(T - 1)
                    didx = px[ox] + dyz[oz * 2 + oy]
                    idx = jnp.where(is_dense, didx, hidx)
                    word = plsc.load_gather(tab_v, [idx])
                    f0 = lax.bitcast_convert_type(word << 16, jnp.float32)
                    f1 = lax.bitcast_convert_type(word & np.int32(-65536),
                                                  jnp.float32)
                    wc = sxy[oy * 2 + ox] * sz[oz]
                    acc0 = acc0 + wc * f0
                    acc1 = acc1 + wc * f1
                e0_v[sl] = acc0
                e1_v[sl] = acc1
                return ()

            lax.fori_loop(0, groups, group_body, (), unroll=False)
            pltpu.sync_copy(e0_v, out.at[pl.ds(2 * level * n + base, STRIPE)])
            pltpu.sync_copy(e1_v,
                            out.at[pl.ds((2 * level + 1) * n + base, STRIPE)])
            return ()

        lax.fori_loop(0, nstripes, stripe_body, (), unroll=False)

    return encode


def _mlp(enc_t, w1t, b1c, w2t, b2c, n, bn=2048):
    def body(e_ref, w1_ref, b1_ref, w2_ref, b2_ref, o_ref):
        e = e_ref[...]
        h = jnp.dot(w1_ref[...], e, preferred_element_type=jnp.float32)
        h = jnp.maximum(h + b1_ref[...], 0.0)
        zz = jnp.dot(w2_ref[...], h, preferred_element_type=jnp.float32)
        zz = zz + b2_ref[...]
        o_ref[...] = 1.0 / (1.0 + jnp.exp(-zz))

    return pl.pallas_call(
        body,
        grid=(n // bn,),
        in_specs=[
            pl.BlockSpec((2 * L, bn), lambda i: (0, i)),
            pl.BlockSpec((64, 2 * L), lambda i: (0, 0)),
            pl.BlockSpec((64, 1), lambda i: (0, 0)),
            pl.BlockSpec((1, 64), lambda i: (0, 0)),
            pl.BlockSpec((1, 1), lambda i: (0, 0)),
        ],
        out_specs=pl.BlockSpec((1, bn), lambda i: (0, i)),
        out_shape=jax.ShapeDtypeStruct((1, n), jnp.float32),
    )(enc_t, w1t, b1c, w2t, b2c)


def kernel(uvi, tables, W1, b1, W2, b2):
    n = uvi.shape[0]
    t16 = tables.astype(jnp.bfloat16)
    bits = lax.bitcast_convert_type(t16, jnp.uint16).astype(jnp.uint32)
    ptab = (bits[..., 0] | (bits[..., 1] << 16)).astype(jnp.int32)
    uvi_t = uvi.T.reshape(3 * n)

    enc_t = _sc_encode(n)(uvi_t, ptab.reshape(L * T)).reshape(2 * L, n)

    mask_t = _mlp(enc_t, W1.T, b1.reshape(64, 1), W2.T, b2.reshape(1, 1), n)
    return mask_t.reshape(n, 1)
